# Initial kernel scaffold; baseline (speedup 1.0000x reference)
import jax
import jax.numpy as jnp
from jax.experimental import pallas as pl


def kernel(x, edge_index, edge_attr, u, batch, W1, b1, W2, b2):
    raise NotImplementedError("write your pallas kernel here")


# speedup vs baseline: 13.9272x; 13.9272x over previous
import dataclasses
import functools

import jax
import jax.numpy as jnp
from jax import lax
from jax.experimental import pallas as pl
from jax.experimental.pallas import tpu as pltpu
from jax.experimental.pallas import tpu_sc as plsc

N_NODES = 10000
N_EDGES = 320000
N_GRAPHS = 64
NODE_DIM = 128
EDGE_DIM = 16
ACC_W = 192
LANES = 16

NW = 32
EDGES_PER_TILE = N_EDGES // NW
E_CHUNK = 2000
E_NCHUNK = EDGES_PER_TILE // E_CHUNK
E_GROUPS = E_CHUNK // LANES

NODE_GROUPS = N_NODES // LANES
GROUPS_PER_TILE = 20
X_CHUNK_G = 5
X_NCHUNK = GROUPS_PER_TILE // X_CHUNK_G
X_CHUNK_ROWS = X_CHUNK_G * LANES


def _sc_pool(x, src, edge_attr, batch):
    mesh = plsc.VectorSubcoreMesh(core_axis_name="c", subcore_axis_name="s")
    cp = pltpu.CompilerParams()
    if "needs_layout_passes" in pltpu.CompilerParams.__dataclass_fields__:
        cp = dataclasses.replace(cp, needs_layout_passes=False)
    if "use_tc_tiling_on_sc" in pltpu.CompilerParams.__dataclass_fields__:
        cp = dataclasses.replace(cp, use_tc_tiling_on_sc=False)

    @functools.partial(
        pl.kernel,
        compiler_params=cp,
        out_type=jax.ShapeDtypeStruct((NW, N_GRAPHS, ACC_W), jnp.float32),
        mesh=mesh,
        scratch_types=[
            pltpu.VMEM((N_NODES,), jnp.int32),
            pltpu.VMEM((N_GRAPHS, ACC_W), jnp.float32),
            pltpu.VMEM((X_CHUNK_ROWS, NODE_DIM), jnp.float32),
            pltpu.VMEM((X_CHUNK_ROWS, NODE_DIM), jnp.float32),
            pltpu.VMEM((E_CHUNK, EDGE_DIM), jnp.float32),
            pltpu.VMEM((E_CHUNK, EDGE_DIM), jnp.float32),
            pltpu.VMEM((E_CHUNK,), jnp.int32),
            pltpu.VMEM((E_CHUNK,), jnp.int32),
            pltpu.SemaphoreType.DMA,
            pltpu.SemaphoreType.DMA,
            pltpu.SemaphoreType.DMA,
            pltpu.SemaphoreType.DMA,
            pltpu.SemaphoreType.DMA,
        ],
    )
    def k(x_hbm, src_hbm, attr_hbm, batch_hbm, out_hbm,
          batch_v, acc, xb0, xb1, ab0, ab1, sb0, sb1,
          sem_batch, sem_x0, sem_x1, sem_e0, sem_e1):
        wid = lax.axis_index("c") * 16 + lax.axis_index("s")

        iota = lax.iota(jnp.int32, LANES)
        ones = jnp.ones((LANES,), jnp.float32)
        zeros = jnp.zeros((LANES,), jnp.float32)
        rots = [(iota + d) & (LANES - 1) for d in range(LANES)]

        xbufs = (xb0, xb1)
        abufs = (ab0, ab1)
        sbufs = (sb0, sb1)
        xsems = (sem_x0, sem_x1)
        esems = (sem_e0, sem_e1)

        h_batch = pltpu.async_copy(batch_hbm, batch_v, sem_batch)

        e_base = wid * EDGES_PER_TILE

        def start_echunk(c, buf):
            ha = pltpu.async_copy(
                attr_hbm.at[pl.ds(e_base + c * E_CHUNK, E_CHUNK)],
                abufs[buf], esems[buf])
            hs = pltpu.async_copy(
                src_hbm.at[pl.ds(e_base + c * E_CHUNK, E_CHUNK)],
                sbufs[buf], esems[buf])
            return ha, hs

        g_base = wid * GROUPS_PER_TILE
        n_groups = jnp.minimum(GROUPS_PER_TILE, NODE_GROUPS - g_base)
        n_xchunks = n_groups // X_CHUNK_G

        def start_xchunk(c, buf):
            row0 = (g_base + c * X_CHUNK_G) * LANES
            return pltpu.async_copy(
                x_hbm.at[pl.ds(row0, X_CHUNK_ROWS)], xbufs[buf], xsems[buf])

        he0 = start_echunk(0, 0)
        hx0 = start_xchunk(0, 0)

        @pl.loop(0, N_GRAPHS)
        def _(r):
            for cg in range(ACC_W // LANES):
                acc[r, pl.ds(cg * LANES, LANES)] = zeros

        h_batch.wait()

        def x_process(c, buf):
            @pl.loop(0, X_CHUNK_G)
            def _(g):
                gg = g_base + c * X_CHUNK_G + g
                node0 = gg * LANES
                b_vec = batch_v[pl.ds(node0, LANES)]
                plsc.addupdate_scatter(acc, [b_vec, iota + 160], ones)
                row0 = g * LANES
                for cg in range(NODE_DIM // LANES):
                    for d in range(LANES):
                        col = cg * LANES + rots[d]
                        vals = plsc.load_gather(
                            xbufs[buf], [row0 + iota, col])
                        plsc.addupdate_scatter(acc, [b_vec, col], vals)

        hx_prev = hx0
        for c in range(X_NCHUNK):
            nxt = None
            if c + 1 < X_NCHUNK:
                @pl.when(c + 1 < n_xchunks)
                def _():
                    start_xchunk(c + 1, (c + 1) % 2)
                nxt = pltpu.make_async_copy(
                    x_hbm.at[pl.ds(0, X_CHUNK_ROWS)],
                    xbufs[(c + 1) % 2], xsems[(c + 1) % 2])

            @pl.when(c < n_xchunks)
            def _():
                hx_prev.wait()
                x_process(c, c % 2)

            hx_prev = nxt

        def e_process(buf):
            @pl.loop(0, E_GROUPS)
            def _(j):
                r0 = j * LANES
                s_vec = sbufs[buf][pl.ds(r0, LANES)]
                b_vec = plsc.load_gather(batch_v, [s_vec])
                plsc.addupdate_scatter(acc, [b_vec, iota + 176], ones)
                for d in range(LANES):
                    vals = plsc.load_gather(abufs[buf], [r0 + iota, rots[d]])
                    plsc.addupdate_scatter(acc, [b_vec, rots[d] + 128], vals)

        he_prev = he0
        for c in range(E_NCHUNK):
            nxt = None
            if c + 1 < E_NCHUNK:
                nxt = start_echunk(c + 1, (c + 1) % 2)
            he_prev[0].wait()
            he_prev[1].wait()
            e_process(c % 2)
            he_prev = nxt

        pltpu.sync_copy(acc, out_hbm.at[wid])

    return k(x, src, edge_attr, batch)


def _tc_head(partials, u, W1, b1, W2, b2):
    def body(p_ref, u_ref, w1_ref, b1_ref, w2_ref, b2_ref, o_ref):
        p = jnp.sum(p_ref[...], axis=0)
        x_sum = p[:, 0:NODE_DIM]
        e_sum = p[:, NODE_DIM:NODE_DIM + EDGE_DIM]
        n_cnt = jnp.sum(p[:, 160:176], axis=1, keepdims=True)
        e_cnt = jnp.sum(p[:, 176:192], axis=1, keepdims=True)
        x_mean = x_sum / jnp.maximum(n_cnt, 1.0)
        e_mean = e_sum / jnp.maximum(e_cnt, 1.0)
        uu = u_ref[...]
        w1 = w1_ref[...]
        h = (
            jnp.dot(uu, w1[0:64], preferred_element_type=jnp.float32)
            + jnp.dot(x_mean, w1[64:192], preferred_element_type=jnp.float32)
            + jnp.dot(e_mean, w1[192:208], preferred_element_type=jnp.float32)
            + b1_ref[...]
        )
        h = jnp.maximum(h, 0.0)
        o_ref[...] = (
            jnp.dot(h, w2_ref[...], preferred_element_type=jnp.float32)
            + b2_ref[...]
        )

    return pl.pallas_call(
        body,
        out_shape=jax.ShapeDtypeStruct((N_GRAPHS, W2.shape[1]), jnp.float32),
    )(partials, u, W1, b1.reshape(1, -1), W2, b2.reshape(1, -1))


def kernel(x, edge_index, edge_attr, u, batch, W1, b1, W2, b2):
    src = edge_index[0].astype(jnp.int32)
    batch32 = batch.astype(jnp.int32)
    partials = _sc_pool(x, src, edge_attr, batch32)
    return _tc_head(partials, u, W1, b1, W2, b2)


# speedup vs baseline: 16.3258x; 1.1722x over previous
import dataclasses
import functools

import jax
import jax.numpy as jnp
from jax import lax
from jax.experimental import pallas as pl
from jax.experimental.pallas import tpu as pltpu
from jax.experimental.pallas import tpu_sc as plsc

N_NODES = 10000
N_EDGES = 320000
N_GRAPHS = 64
NODE_DIM = 128
EDGE_DIM = 16
ACC_ROWS = 128
LANES = 16

NW = 32
EDGES_PER_TILE = N_EDGES // NW
E_CHUNK = 2000
E_NCHUNK = EDGES_PER_TILE // E_CHUNK
E_GROUPS = E_CHUNK // LANES
E_CHUNK_ROWS = E_CHUNK * EDGE_DIM // NODE_DIM

NODE_GROUPS = N_NODES // LANES
GROUPS_PER_TILE = 20
X_CHUNK_G = 5
X_NCHUNK = GROUPS_PER_TILE // X_CHUNK_G
X_CHUNK_ROWS = X_CHUNK_G * LANES


def _sc_pool(x, src, attr_rm, batch):
    mesh = plsc.VectorSubcoreMesh(core_axis_name="c", subcore_axis_name="s")
    cp = pltpu.CompilerParams()
    if "needs_layout_passes" in pltpu.CompilerParams.__dataclass_fields__:
        cp = dataclasses.replace(cp, needs_layout_passes=False)
    if "use_tc_tiling_on_sc" in pltpu.CompilerParams.__dataclass_fields__:
        cp = dataclasses.replace(cp, use_tc_tiling_on_sc=False)

    @functools.partial(
        pl.kernel,
        out_type=jax.ShapeDtypeStruct((NW, ACC_ROWS, NODE_DIM), jnp.float32),
        mesh=mesh,
        compiler_params=cp,
        scratch_types=[
            pltpu.VMEM((N_NODES,), jnp.int32),
            pltpu.VMEM((ACC_ROWS, NODE_DIM), jnp.float32),
            pltpu.VMEM((X_CHUNK_ROWS, NODE_DIM), jnp.float32),
            pltpu.VMEM((X_CHUNK_ROWS, NODE_DIM), jnp.float32),
            pltpu.VMEM((E_CHUNK_ROWS, NODE_DIM), jnp.float32),
            pltpu.VMEM((E_CHUNK_ROWS, NODE_DIM), jnp.float32),
            pltpu.VMEM((E_CHUNK,), jnp.int32),
            pltpu.VMEM((E_CHUNK,), jnp.int32),
            pltpu.SemaphoreType.DMA,
            pltpu.SemaphoreType.DMA,
            pltpu.SemaphoreType.DMA,
            pltpu.SemaphoreType.DMA,
            pltpu.SemaphoreType.DMA,
        ],
    )
    def k(x_hbm, src_hbm, attr_hbm, batch_hbm, out_hbm,
          batch_v, acc, xb0, xb1, ab0, ab1, sb0, sb1,
          sem_batch, sem_x0, sem_x1, sem_e0, sem_e1):
        wid = lax.axis_index("c") * 16 + lax.axis_index("s")

        iota = lax.iota(jnp.int32, LANES)
        ones = jnp.ones((LANES,), jnp.float32)
        zeros = jnp.zeros((LANES,), jnp.float32)
        rots = [(iota + d) & (LANES - 1) for d in range(LANES)]

        xbufs = (xb0, xb1)
        abufs = (ab0, ab1)
        sbufs = (sb0, sb1)
        xsems = (sem_x0, sem_x1)
        esems = (sem_e0, sem_e1)

        h_batch = pltpu.async_copy(batch_hbm, batch_v, sem_batch)

        e_base = wid * EDGES_PER_TILE
        er_base = wid * (EDGES_PER_TILE * EDGE_DIM // NODE_DIM)

        def start_echunk(c, buf):
            ha = pltpu.async_copy(
                attr_hbm.at[pl.ds(er_base + c * E_CHUNK_ROWS, E_CHUNK_ROWS)],
                abufs[buf], esems[buf])
            hs = pltpu.async_copy(
                src_hbm.at[pl.ds(e_base + c * E_CHUNK, E_CHUNK)],
                sbufs[buf], esems[buf])
            return ha, hs

        g_base = wid * GROUPS_PER_TILE
        n_groups = jnp.minimum(GROUPS_PER_TILE, NODE_GROUPS - g_base)
        n_xchunks = n_groups // X_CHUNK_G

        def start_xchunk(c, buf):
            row0 = (g_base + c * X_CHUNK_G) * LANES
            return pltpu.async_copy(
                x_hbm.at[pl.ds(row0, X_CHUNK_ROWS)], xbufs[buf], xsems[buf])

        he0 = start_echunk(0, 0)
        hx0 = start_xchunk(0, 0)

        @pl.loop(0, ACC_ROWS)
        def _(r):
            for cg in range(NODE_DIM // LANES):
                acc[r, pl.ds(cg * LANES, LANES)] = zeros

        h_batch.wait()

        def x_process(c, buf):
            @pl.loop(0, X_CHUNK_G)
            def _(g):
                gg = g_base + c * X_CHUNK_G + g
                node0 = gg * LANES
                b_vec = batch_v[pl.ds(node0, LANES)]
                plsc.addupdate_scatter(acc, [b_vec + 64, iota + 16], ones)
                row0 = g * LANES
                for cg in range(NODE_DIM // LANES):
                    vals = [
                        plsc.load_gather(
                            xbufs[buf], [row0 + iota, cg * LANES + rots[d]])
                        for d in range(LANES)
                    ]
                    for d in range(LANES):
                        plsc.addupdate_scatter(
                            acc, [b_vec, cg * LANES + rots[d]], vals[d])

        hx_prev = hx0
        for c in range(X_NCHUNK):
            nxt = None
            if c + 1 < X_NCHUNK:
                @pl.when(c + 1 < n_xchunks)
                def _():
                    start_xchunk(c + 1, (c + 1) % 2)
                nxt = pltpu.make_async_copy(
                    x_hbm.at[pl.ds(0, X_CHUNK_ROWS)],
                    xbufs[(c + 1) % 2], xsems[(c + 1) % 2])

            @pl.when(c < n_xchunks)
            def _():
                hx_prev.wait()
                x_process(c, c % 2)

            hx_prev = nxt

        def e_process(buf):
            @pl.loop(0, E_GROUPS)
            def _(j):
                r0 = j * LANES
                e_loc = r0 + iota
                row_e = e_loc >> 3
                col_e = (e_loc & 7) << 4
                s_vec = sbufs[buf][pl.ds(r0, LANES)]
                b_vec = plsc.load_gather(batch_v, [s_vec])
                br = b_vec + 64
                plsc.addupdate_scatter(acc, [br, iota + 32], ones)
                vals = [
                    plsc.load_gather(abufs[buf], [row_e, col_e + rots[d]])
                    for d in range(LANES)
                ]
                for d in range(LANES):
                    plsc.addupdate_scatter(acc, [br, rots[d]], vals[d])

        he_prev = he0
        for c in range(E_NCHUNK):
            nxt = None
            if c + 1 < E_NCHUNK:
                nxt = start_echunk(c + 1, (c + 1) % 2)
            he_prev[0].wait()
            he_prev[1].wait()
            e_process(c % 2)
            he_prev = nxt

        pltpu.sync_copy(acc, out_hbm.at[wid])

    return k(x, src, attr_rm, batch)


def _tc_head(partials, u, W1, b1, W2, b2):
    def body(p_ref, u_ref, w1_ref, b1_ref, w2_ref, b2_ref, o_ref):
        p = jnp.sum(p_ref[...], axis=0)
        x_sum = p[0:64, :]
        q = p[64:128, :]
        e_sum = q[:, 0:16]
        n_cnt = jnp.sum(q[:, 16:32], axis=1, keepdims=True)
        e_cnt = jnp.sum(q[:, 32:48], axis=1, keepdims=True)
        x_mean = x_sum / jnp.maximum(n_cnt, 1.0)
        e_mean = e_sum / jnp.maximum(e_cnt, 1.0)
        uu = u_ref[...]
        w1 = w1_ref[...]
        h = (
            jnp.dot(uu, w1[0:64], preferred_element_type=jnp.float32)
            + jnp.dot(x_mean, w1[64:192], preferred_element_type=jnp.float32)
            + jnp.dot(e_mean, w1[192:208], preferred_element_type=jnp.float32)
            + b1_ref[...]
        )
        h = jnp.maximum(h, 0.0)
        o_ref[...] = (
            jnp.dot(h, w2_ref[...], preferred_element_type=jnp.float32)
            + b2_ref[...]
        )

    return pl.pallas_call(
        body,
        out_shape=jax.ShapeDtypeStruct((N_GRAPHS, W2.shape[1]), jnp.float32),
    )(partials, u, W1, b1.reshape(1, -1), W2, b2.reshape(1, -1))


def kernel(x, edge_index, edge_attr, u, batch, W1, b1, W2, b2):
    src = edge_index[0].astype(jnp.int32)
    batch32 = batch.astype(jnp.int32)
    attr_rm = edge_attr.reshape(N_EDGES * EDGE_DIM // NODE_DIM, NODE_DIM)
    partials = _sc_pool(x, src, attr_rm, batch32)
    return _tc_head(partials, u, W1, b1, W2, b2)


# speedup vs baseline: 35.0747x; 2.1484x over previous
import dataclasses
import functools

import jax
import jax.numpy as jnp
from jax import lax
from jax.experimental import pallas as pl
from jax.experimental.pallas import tpu as pltpu
from jax.experimental.pallas import tpu_sc as plsc

N_NODES = 10000
N_EDGES = 320000
N_GRAPHS = 64
NODE_DIM = 128
EDGE_DIM = 16
ACC_ROWS = 128
LANES = 16

NW = 32
EDGES_PER_TILE = N_EDGES // NW
E_CHUNK = 2000
E_NCHUNK = EDGES_PER_TILE // E_CHUNK
E_GROUPS = E_CHUNK // LANES

NODE_GROUPS = N_NODES // LANES
GROUPS_PER_TILE = 20
X_CHUNK_G = 5
X_NCHUNK = GROUPS_PER_TILE // X_CHUNK_G
X_CHUNK_ROWS = X_CHUNK_G * LANES


def _sc_pool(x, src, attr_rm, batch):
    mesh = plsc.VectorSubcoreMesh(core_axis_name="c", subcore_axis_name="s")
    cp = pltpu.CompilerParams()
    if "needs_layout_passes" in pltpu.CompilerParams.__dataclass_fields__:
        cp = dataclasses.replace(cp, needs_layout_passes=False)
    if "use_tc_tiling_on_sc" in pltpu.CompilerParams.__dataclass_fields__:
        cp = dataclasses.replace(cp, use_tc_tiling_on_sc=False)

    @functools.partial(
        pl.kernel,
        out_type=jax.ShapeDtypeStruct((NW, ACC_ROWS, NODE_DIM), jnp.float32),
        mesh=mesh,
        compiler_params=cp,
        scratch_types=[
            pltpu.VMEM((N_NODES,), jnp.int32),
            pltpu.VMEM((ACC_ROWS, NODE_DIM), jnp.float32),
            pltpu.VMEM((X_CHUNK_ROWS, NODE_DIM), jnp.float32),
            pltpu.VMEM((X_CHUNK_ROWS, NODE_DIM), jnp.float32),
            pltpu.VMEM((EDGE_DIM, E_CHUNK), jnp.float32),
            pltpu.VMEM((EDGE_DIM, E_CHUNK), jnp.float32),
            pltpu.VMEM((E_CHUNK,), jnp.int32),
            pltpu.VMEM((E_CHUNK,), jnp.int32),
            pltpu.SemaphoreType.DMA,
            pltpu.SemaphoreType.DMA,
            pltpu.SemaphoreType.DMA,
            pltpu.SemaphoreType.DMA,
            pltpu.SemaphoreType.DMA,
        ],
    )
    def k(x_hbm, src_hbm, attr_hbm, batch_hbm, out_hbm,
          batch_v, acc, xb0, xb1, ab0, ab1, sb0, sb1,
          sem_batch, sem_x0, sem_x1, sem_e0, sem_e1):
        wid = lax.axis_index("c") * 16 + lax.axis_index("s")

        iota = lax.iota(jnp.int32, LANES)
        ones = jnp.ones((LANES,), jnp.float32)
        zeros = jnp.zeros((LANES,), jnp.float32)
        rots = [(iota + d) & (LANES - 1) for d in range(LANES)]

        xbufs = (xb0, xb1)
        abufs = (ab0, ab1)
        sbufs = (sb0, sb1)
        xsems = (sem_x0, sem_x1)
        esems = (sem_e0, sem_e1)

        h_batch = pltpu.async_copy(batch_hbm, batch_v, sem_batch)

        e_base = wid * EDGES_PER_TILE

        def start_echunk(c, buf):
            ha = pltpu.async_copy(
                attr_hbm.at[:, pl.ds(e_base + c * E_CHUNK, E_CHUNK)],
                abufs[buf], esems[buf])
            hs = pltpu.async_copy(
                src_hbm.at[pl.ds(e_base + c * E_CHUNK, E_CHUNK)],
                sbufs[buf], esems[buf])
            return ha, hs

        g_base = wid * GROUPS_PER_TILE
        n_groups = jnp.minimum(GROUPS_PER_TILE, NODE_GROUPS - g_base)
        n_xchunks = n_groups // X_CHUNK_G

        def start_xchunk(c, buf):
            row0 = (g_base + c * X_CHUNK_G) * LANES
            return pltpu.async_copy(
                x_hbm.at[pl.ds(row0, X_CHUNK_ROWS)], xbufs[buf], xsems[buf])

        he0 = start_echunk(0, 0)
        hx0 = start_xchunk(0, 0)

        @pl.loop(0, ACC_ROWS)
        def _(r):
            for cg in range(NODE_DIM // LANES):
                acc[r, pl.ds(cg * LANES, LANES)] = zeros

        h_batch.wait()

        def x_process(c, buf):
            @pl.loop(0, X_CHUNK_G)
            def _(g):
                gg = g_base + c * X_CHUNK_G + g
                node0 = gg * LANES
                b_vec = batch_v[pl.ds(node0, LANES)]
                plsc.addupdate_scatter(acc, [b_vec + 64, iota + 16], ones)
                row0 = g * LANES
                for cg in range(NODE_DIM // LANES):
                    vals = [
                        plsc.load_gather(
                            xbufs[buf], [row0 + iota, cg * LANES + rots[d]])
                        for d in range(LANES)
                    ]
                    for d in range(LANES):
                        plsc.addupdate_scatter(
                            acc, [b_vec, cg * LANES + rots[d]], vals[d])

        hx_prev = hx0
        for c in range(X_NCHUNK):
            nxt = None
            if c + 1 < X_NCHUNK:
                @pl.when(c + 1 < n_xchunks)
                def _():
                    start_xchunk(c + 1, (c + 1) % 2)
                nxt = pltpu.make_async_copy(
                    x_hbm.at[pl.ds(0, X_CHUNK_ROWS)],
                    xbufs[(c + 1) % 2], xsems[(c + 1) % 2])

            @pl.when(c < n_xchunks)
            def _():
                hx_prev.wait()
                x_process(c, c % 2)

            hx_prev = nxt

        def e_process(buf):
            @pl.loop(0, E_GROUPS)
            def _(j):
                r0 = j * LANES
                e_loc = r0 + iota
                s_vec = sbufs[buf][pl.ds(r0, LANES)]
                b_vec = plsc.load_gather(batch_v, [s_vec])
                br = b_vec + 64
                plsc.addupdate_scatter(acc, [br, iota + 32], ones)
                vals = [
                    plsc.load_gather(abufs[buf], [rots[d], e_loc])
                    for d in range(LANES)
                ]
                for d in range(LANES):
                    plsc.addupdate_scatter(acc, [br, rots[d]], vals[d])

        he_prev = he0
        for c in range(E_NCHUNK):
            nxt = None
            if c + 1 < E_NCHUNK:
                nxt = start_echunk(c + 1, (c + 1) % 2)
            he_prev[0].wait()
            he_prev[1].wait()
            e_process(c % 2)
            he_prev = nxt

        pltpu.sync_copy(acc, out_hbm.at[wid])

    return k(x, src, attr_rm, batch)


def _tc_head(partials, u, W1, b1, W2, b2):
    def body(p_ref, u_ref, w1_ref, b1_ref, w2_ref, b2_ref, o_ref):
        p = jnp.sum(p_ref[...], axis=0)
        x_sum = p[0:64, :]
        q = p[64:128, :]
        e_sum = q[:, 0:16]
        n_cnt = jnp.sum(q[:, 16:32], axis=1, keepdims=True)
        e_cnt = jnp.sum(q[:, 32:48], axis=1, keepdims=True)
        x_mean = x_sum / jnp.maximum(n_cnt, 1.0)
        e_mean = e_sum / jnp.maximum(e_cnt, 1.0)
        uu = u_ref[...]
        w1 = w1_ref[...]
        h = (
            jnp.dot(uu, w1[0:64], preferred_element_type=jnp.float32)
            + jnp.dot(x_mean, w1[64:192], preferred_element_type=jnp.float32)
            + jnp.dot(e_mean, w1[192:208], preferred_element_type=jnp.float32)
            + b1_ref[...]
        )
        h = jnp.maximum(h, 0.0)
        o_ref[...] = (
            jnp.dot(h, w2_ref[...], preferred_element_type=jnp.float32)
            + b2_ref[...]
        )

    return pl.pallas_call(
        body,
        out_shape=jax.ShapeDtypeStruct((N_GRAPHS, W2.shape[1]), jnp.float32),
    )(partials, u, W1, b1.reshape(1, -1), W2, b2.reshape(1, -1))


def kernel(x, edge_index, edge_attr, u, batch, W1, b1, W2, b2):
    src = edge_index[0].astype(jnp.int32)
    batch32 = batch.astype(jnp.int32)
    attr_t = edge_attr.T
    partials = _sc_pool(x, src, attr_t, batch32)
    return _tc_head(partials, u, W1, b1, W2, b2)


# speedup vs baseline: 50.1513x; 1.4298x over previous
import dataclasses
import functools

import jax
import jax.numpy as jnp
from jax import lax
from jax.experimental import pallas as pl
from jax.experimental.pallas import tpu as pltpu
from jax.experimental.pallas import tpu_sc as plsc

N_NODES = 10000
N_EDGES = 320000
N_GRAPHS = 64
NODE_DIM = 128
EDGE_DIM = 16
ACC_ROWS = 128
LANES = 16

NW = 32
N_CT = N_EDGES // 128
CT_BASE_PER_TILE = N_CT // NW
CT_EXTRA = N_CT - CT_BASE_PER_TILE * NW
CT_CHUNK = 16
E_NCHUNK = (CT_BASE_PER_TILE + 1 + CT_CHUNK - 1) // CT_CHUNK

NODE_GROUPS = N_NODES // LANES
GROUPS_PER_TILE = 20
X_CHUNK_G = 5
X_NCHUNK = GROUPS_PER_TILE // X_CHUNK_G
X_CHUNK_ROWS = X_CHUNK_G * LANES


def _sc_pool(x, e3, a4, batch):
    mesh = plsc.VectorSubcoreMesh(core_axis_name="c", subcore_axis_name="s")
    cp = pltpu.CompilerParams()
    if "needs_layout_passes" in pltpu.CompilerParams.__dataclass_fields__:
        cp = dataclasses.replace(cp, needs_layout_passes=False)
    if "use_tc_tiling_on_sc" in pltpu.CompilerParams.__dataclass_fields__:
        cp = dataclasses.replace(cp, use_tc_tiling_on_sc=False)

    @functools.partial(
        pl.kernel,
        out_type=jax.ShapeDtypeStruct((NW, ACC_ROWS, NODE_DIM), jnp.float32),
        mesh=mesh,
        compiler_params=cp,
        scratch_types=[
            pltpu.VMEM((N_NODES,), jnp.int32),
            pltpu.VMEM((ACC_ROWS, NODE_DIM), jnp.float32),
            pltpu.VMEM((X_CHUNK_ROWS, NODE_DIM), jnp.float32),
            pltpu.VMEM((X_CHUNK_ROWS, NODE_DIM), jnp.float32),
            pltpu.VMEM((2, CT_CHUNK, 8, NODE_DIM), jnp.float32),
            pltpu.VMEM((2, CT_CHUNK, 8, NODE_DIM), jnp.float32),
            pltpu.VMEM((CT_CHUNK, NODE_DIM), jnp.int32),
            pltpu.VMEM((CT_CHUNK, NODE_DIM), jnp.int32),
            pltpu.SemaphoreType.DMA,
            pltpu.SemaphoreType.DMA,
            pltpu.SemaphoreType.DMA,
            pltpu.SemaphoreType.DMA,
            pltpu.SemaphoreType.DMA,
        ],
    )
    def k(x_hbm, e3_hbm, a4_hbm, batch_hbm, out_hbm,
          batch_v, acc, xb0, xb1, ab0, ab1, sb0, sb1,
          sem_batch, sem_x0, sem_x1, sem_e0, sem_e1):
        wid = lax.axis_index("c") * 16 + lax.axis_index("s")

        iota = lax.iota(jnp.int32, LANES)
        ones = jnp.ones((LANES,), jnp.float32)
        zeros = jnp.zeros((LANES,), jnp.float32)
        ones_i = jnp.ones((LANES,), jnp.int32)
        rots = [(iota + d) & (LANES - 1) for d in range(LANES)]

        xbufs = (xb0, xb1)
        abufs = (ab0, ab1)
        sbufs = (sb0, sb1)
        xsems = (sem_x0, sem_x1)
        esems = (sem_e0, sem_e1)

        h_batch = pltpu.async_copy(batch_hbm, batch_v, sem_batch)

        ct_base = wid * CT_BASE_PER_TILE + jnp.minimum(wid, CT_EXTRA)
        n_ct = CT_BASE_PER_TILE + jnp.where(wid < CT_EXTRA, 1, 0)
        n_echunks = (n_ct + CT_CHUNK - 1) // CT_CHUNK

        def echunk_start_ct(c):
            return ct_base + jnp.minimum(c * CT_CHUNK, n_ct - CT_CHUNK)

        def start_echunk(c, buf):
            ct0 = echunk_start_ct(c)
            ha = pltpu.async_copy(
                a4_hbm.at[:, pl.ds(ct0, CT_CHUNK)], abufs[buf], esems[buf])
            hs = pltpu.async_copy(
                e3_hbm.at[pl.ds(ct0, CT_CHUNK), 0], sbufs[buf], esems[buf])
            return ha, hs

        g_base = wid * GROUPS_PER_TILE
        n_groups = jnp.minimum(GROUPS_PER_TILE, NODE_GROUPS - g_base)
        n_xchunks = n_groups // X_CHUNK_G

        def start_xchunk(c, buf):
            row0 = (g_base + c * X_CHUNK_G) * LANES
            return pltpu.async_copy(
                x_hbm.at[pl.ds(row0, X_CHUNK_ROWS)], xbufs[buf], xsems[buf])

        he0 = start_echunk(0, 0)
        hx0 = start_xchunk(0, 0)

        @pl.loop(0, ACC_ROWS)
        def _(r):
            for cg in range(NODE_DIM // LANES):
                acc[r, pl.ds(cg * LANES, LANES)] = zeros

        h_batch.wait()

        def x_process(c, buf):
            @pl.loop(0, X_CHUNK_G)
            def _(g):
                gg = g_base + c * X_CHUNK_G + g
                node0 = gg * LANES
                b_vec = batch_v[pl.ds(node0, LANES)]
                plsc.addupdate_scatter(acc, [b_vec + 64, iota + 16], ones)
                row0 = g * LANES
                for cg in range(NODE_DIM // LANES):
                    vals = [
                        plsc.load_gather(
                            xbufs[buf], [row0 + iota, cg * LANES + rots[d]])
                        for d in range(LANES)
                    ]
                    for d in range(LANES):
                        plsc.addupdate_scatter(
                            acc, [b_vec, cg * LANES + rots[d]], vals[d])

        hx_prev = hx0
        for c in range(X_NCHUNK):
            nxt = None
            if c + 1 < X_NCHUNK:
                @pl.when(c + 1 < n_xchunks)
                def _():
                    start_xchunk(c + 1, (c + 1) % 2)
                nxt = pltpu.make_async_copy(
                    x_hbm.at[pl.ds(0, X_CHUNK_ROWS)],
                    xbufs[(c + 1) % 2], xsems[(c + 1) % 2])

            @pl.when(c < n_xchunks)
            def _():
                hx_prev.wait()
                x_process(c, c % 2)

            hx_prev = nxt

        def e_process(c, buf):
            glo = (c * CT_CHUNK - (echunk_start_ct(c) - ct_base)) * 8

            @pl.loop(glo, CT_CHUNK * 8)
            def _(g):
                ct_l = g >> 3
                lb = (g & 7) << 4
                ct_v = ct_l * ones_i
                ln_v = lb + iota
                s_vec = sbufs[buf][ct_l, pl.ds(lb, LANES)]
                b_vec = plsc.load_gather(batch_v, [s_vec])
                br = b_vec + 64
                plsc.addupdate_scatter(acc, [br, iota + 32], ones)
                vals = [
                    plsc.load_gather(
                        abufs[buf],
                        [rots[d] >> 3, ct_v, rots[d] & 7, ln_v])
                    for d in range(LANES)
                ]
                for d in range(LANES):
                    plsc.addupdate_scatter(acc, [br, rots[d]], vals[d])

        he_prev = he0
        for c in range(E_NCHUNK):
            nxt = None
            if c + 1 < E_NCHUNK:
                @pl.when(c + 1 < n_echunks)
                def _():
                    start_echunk(c + 1, (c + 1) % 2)
                nxt = (
                    pltpu.make_async_copy(
                        a4_hbm.at[:, pl.ds(0, CT_CHUNK)],
                        abufs[(c + 1) % 2], esems[(c + 1) % 2]),
                    pltpu.make_async_copy(
                        e3_hbm.at[pl.ds(0, CT_CHUNK), 0],
                        sbufs[(c + 1) % 2], esems[(c + 1) % 2]),
                )

            @pl.when(c < n_echunks)
            def _():
                he_prev[0].wait()
                he_prev[1].wait()
                e_process(c, c % 2)

            he_prev = nxt

        pltpu.sync_copy(acc, out_hbm.at[wid])

    return k(x, e3, a4, batch)


def _tc_head(partials, u, W1, b1, W2, b2):
    def body(p_ref, u_ref, w1_ref, b1_ref, w2_ref, b2_ref, o_ref):
        p = jnp.sum(p_ref[...], axis=0)
        x_sum = p[0:64, :]
        q = p[64:128, :]
        e_sum = q[:, 0:16]
        n_cnt = jnp.sum(q[:, 16:32], axis=1, keepdims=True)
        e_cnt = jnp.sum(q[:, 32:48], axis=1, keepdims=True)
        x_mean = x_sum / jnp.maximum(n_cnt, 1.0)
        e_mean = e_sum / jnp.maximum(e_cnt, 1.0)
        uu = u_ref[...]
        w1 = w1_ref[...]
        h = (
            jnp.dot(uu, w1[0:64], preferred_element_type=jnp.float32)
            + jnp.dot(x_mean, w1[64:192], preferred_element_type=jnp.float32)
            + jnp.dot(e_mean, w1[192:208], preferred_element_type=jnp.float32)
            + b1_ref[...]
        )
        h = jnp.maximum(h, 0.0)
        o_ref[...] = (
            jnp.dot(h, w2_ref[...], preferred_element_type=jnp.float32)
            + b2_ref[...]
        )

    return pl.pallas_call(
        body,
        out_shape=jax.ShapeDtypeStruct((N_GRAPHS, W2.shape[1]), jnp.float32),
    )(partials, u, W1, b1.reshape(1, -1), W2, b2.reshape(1, -1))


def kernel(x, edge_index, edge_attr, u, batch, W1, b1, W2, b2):
    batch32 = batch.astype(jnp.int32)
    e3 = edge_index.astype(jnp.int32).reshape(2, N_CT, 128).transpose(1, 0, 2)
    a4 = (edge_attr.T.reshape(EDGE_DIM // 8, 8, N_CT, 128)
          .transpose(0, 2, 1, 3))
    partials = _sc_pool(x, e3, a4, batch32)
    return _tc_head(partials, u, W1, b1, W2, b2)


# speedup vs baseline: 61.3157x; 1.2226x over previous
import dataclasses
import functools

import jax
import jax.numpy as jnp
from jax import lax
from jax.experimental import pallas as pl
from jax.experimental.pallas import tpu as pltpu
from jax.experimental.pallas import tpu_sc as plsc

N_NODES = 10000
N_EDGES = 320000
N_GRAPHS = 64
NODE_DIM = 128
EDGE_DIM = 16
ACC_ROWS = 64
LANES = 16

NW = 32
N_CT = N_EDGES // 128
CT_BASE_PER_TILE = N_CT // NW
CT_EXTRA = N_CT - CT_BASE_PER_TILE * NW
CT_CHUNK = 16
E_NCHUNK = (CT_BASE_PER_TILE + 1 + CT_CHUNK - 1) // CT_CHUNK

N_PAD_BLOCKS = 80


def _sc_edge_pool(e3, a4, batch):
    mesh = plsc.VectorSubcoreMesh(core_axis_name="c", subcore_axis_name="s")
    cp = pltpu.CompilerParams()
    if "needs_layout_passes" in pltpu.CompilerParams.__dataclass_fields__:
        cp = dataclasses.replace(cp, needs_layout_passes=False)
    if "use_tc_tiling_on_sc" in pltpu.CompilerParams.__dataclass_fields__:
        cp = dataclasses.replace(cp, use_tc_tiling_on_sc=False)

    @functools.partial(
        pl.kernel,
        out_type=jax.ShapeDtypeStruct((NW, ACC_ROWS, NODE_DIM), jnp.float32),
        mesh=mesh,
        compiler_params=cp,
        scratch_types=[
            pltpu.VMEM((N_NODES,), jnp.int32),
            pltpu.VMEM((ACC_ROWS, NODE_DIM), jnp.float32),
            pltpu.VMEM((2, CT_CHUNK, 8, NODE_DIM), jnp.float32),
            pltpu.VMEM((2, CT_CHUNK, 8, NODE_DIM), jnp.float32),
            pltpu.VMEM((CT_CHUNK, NODE_DIM), jnp.int32),
            pltpu.VMEM((CT_CHUNK, NODE_DIM), jnp.int32),
            pltpu.SemaphoreType.DMA,
            pltpu.SemaphoreType.DMA,
            pltpu.SemaphoreType.DMA,
        ],
    )
    def k(e3_hbm, a4_hbm, batch_hbm, out_hbm,
          batch_v, acc, ab0, ab1, sb0, sb1,
          sem_batch, sem_e0, sem_e1):
        wid = lax.axis_index("c") * 16 + lax.axis_index("s")

        iota = lax.iota(jnp.int32, LANES)
        ones = jnp.ones((LANES,), jnp.float32)
        zeros = jnp.zeros((LANES,), jnp.float32)
        ones_i = jnp.ones((LANES,), jnp.int32)
        rots = [(iota + d) & (LANES - 1) for d in range(LANES)]

        abufs = (ab0, ab1)
        sbufs = (sb0, sb1)
        esems = (sem_e0, sem_e1)

        h_batch = pltpu.async_copy(batch_hbm, batch_v, sem_batch)

        ct_base = wid * CT_BASE_PER_TILE + jnp.minimum(wid, CT_EXTRA)
        n_ct = CT_BASE_PER_TILE + jnp.where(wid < CT_EXTRA, 1, 0)
        n_echunks = (n_ct + CT_CHUNK - 1) // CT_CHUNK

        def echunk_start_ct(c):
            return ct_base + jnp.minimum(c * CT_CHUNK, n_ct - CT_CHUNK)

        def start_echunk(c, buf):
            ct0 = echunk_start_ct(c)
            ha = pltpu.async_copy(
                a4_hbm.at[:, pl.ds(ct0, CT_CHUNK)], abufs[buf], esems[buf])
            hs = pltpu.async_copy(
                e3_hbm.at[pl.ds(ct0, CT_CHUNK), 0], sbufs[buf], esems[buf])
            return ha, hs

        he0 = start_echunk(0, 0)

        @pl.loop(0, ACC_ROWS)
        def _(r):
            for cg in range(NODE_DIM // LANES):
                acc[r, pl.ds(cg * LANES, LANES)] = zeros

        h_batch.wait()

        def e_process(c, buf):
            glo = (c * CT_CHUNK - (echunk_start_ct(c) - ct_base)) * 8

            @pl.loop(glo, CT_CHUNK * 8)
            def _(g):
                ct_l = g >> 3
                lb = (g & 7) << 4
                ct_v = ct_l * ones_i
                ln_v = lb + iota
                s_vec = sbufs[buf][ct_l, pl.ds(lb, LANES)]
                b_vec = plsc.load_gather(batch_v, [s_vec])
                plsc.addupdate_scatter(acc, [b_vec, iota + 32], ones)
                vals = [
                    plsc.load_gather(
                        abufs[buf],
                        [rots[d] >> 3, ct_v, rots[d] & 7, ln_v])
                    for d in range(LANES)
                ]
                for d in range(LANES):
                    plsc.addupdate_scatter(acc, [b_vec, rots[d]], vals[d])

        he_prev = he0
        for c in range(E_NCHUNK):
            nxt = None
            if c + 1 < E_NCHUNK:
                @pl.when(c + 1 < n_echunks)
                def _():
                    start_echunk(c + 1, (c + 1) % 2)
                nxt = (
                    pltpu.make_async_copy(
                        a4_hbm.at[:, pl.ds(0, CT_CHUNK)],
                        abufs[(c + 1) % 2], esems[(c + 1) % 2]),
                    pltpu.make_async_copy(
                        e3_hbm.at[pl.ds(0, CT_CHUNK), 0],
                        sbufs[(c + 1) % 2], esems[(c + 1) % 2]),
                )

            @pl.when(c < n_echunks)
            def _():
                he_prev[0].wait()
                he_prev[1].wait()
                e_process(c, c % 2)

            he_prev = nxt

        pltpu.sync_copy(acc, out_hbm.at[wid])

    return k(e3, a4, batch)


def _tc_node_pool(x, batch_pad):
    def body(x_ref, b_ref, xs_ref, nc_ref):
        gi = lax.broadcasted_iota(jnp.int32, (N_GRAPHS, NODE_DIM), 0)

        def step(bk, carry):
            xs, nc = carry
            bb = b_ref[bk, :].reshape(1, NODE_DIM)
            oh = (bb == gi).astype(jnp.float32)
            xblk = x_ref[pl.ds(bk * NODE_DIM, NODE_DIM), :]
            xs = xs + jnp.dot(oh, xblk, preferred_element_type=jnp.float32)
            return xs, nc + oh

        xs0 = jnp.zeros((N_GRAPHS, NODE_DIM), jnp.float32)
        xs, nc = lax.fori_loop(0, N_PAD_BLOCKS, step, (xs0, xs0))
        xs_ref[...] = xs
        nc_ref[...] = nc

    return pl.pallas_call(
        body,
        out_shape=(
            jax.ShapeDtypeStruct((N_GRAPHS, NODE_DIM), jnp.float32),
            jax.ShapeDtypeStruct((N_GRAPHS, NODE_DIM), jnp.float32),
        ),
    )(x, batch_pad)


def _tc_head(partials, xs, nc, u, W1, b1, W2, b2):
    def body(p_ref, xs_ref, nc_ref, u_ref, w1_ref, b1_ref, w2_ref, b2_ref,
             o_ref):
        p = jnp.sum(p_ref[...], axis=0)
        e_sum = p[:, 0:16]
        e_cnt = jnp.sum(p[:, 32:48], axis=1, keepdims=True)
        n_cnt = jnp.sum(nc_ref[...], axis=1, keepdims=True)
        x_mean = xs_ref[...] / jnp.maximum(n_cnt, 1.0)
        e_mean = e_sum / jnp.maximum(e_cnt, 1.0)
        uu = u_ref[...]
        w1 = w1_ref[...]
        h = (
            jnp.dot(uu, w1[0:64], preferred_element_type=jnp.float32)
            + jnp.dot(x_mean, w1[64:192], preferred_element_type=jnp.float32)
            + jnp.dot(e_mean, w1[192:208], preferred_element_type=jnp.float32)
            + b1_ref[...]
        )
        h = jnp.maximum(h, 0.0)
        o_ref[...] = (
            jnp.dot(h, w2_ref[...], preferred_element_type=jnp.float32)
            + b2_ref[...]
        )

    return pl.pallas_call(
        body,
        out_shape=jax.ShapeDtypeStruct((N_GRAPHS, W2.shape[1]), jnp.float32),
    )(partials, xs, nc, u, W1, b1.reshape(1, -1), W2, b2.reshape(1, -1))


def kernel(x, edge_index, edge_attr, u, batch, W1, b1, W2, b2):
    batch32 = batch.astype(jnp.int32)
    e3 = edge_index.astype(jnp.int32).reshape(2, N_CT, 128).transpose(1, 0, 2)
    a4 = (edge_attr.T.reshape(EDGE_DIM // 8, 8, N_CT, 128)
          .transpose(0, 2, 1, 3))
    batch_pad = jnp.pad(
        batch32, (0, N_PAD_BLOCKS * NODE_DIM - N_NODES),
        constant_values=N_GRAPHS).reshape(N_PAD_BLOCKS, NODE_DIM)
    partials = _sc_edge_pool(e3, a4, batch32)
    xs, nc = _tc_node_pool(x, batch_pad)
    return _tc_head(partials, xs, nc, u, W1, b1, W2, b2)


# speedup vs baseline: 61.7862x; 1.0077x over previous
import dataclasses
import functools

import jax
import jax.numpy as jnp
from jax import lax
from jax.experimental import pallas as pl
from jax.experimental.pallas import tpu as pltpu
from jax.experimental.pallas import tpu_sc as plsc

N_NODES = 10000
N_EDGES = 320000
N_GRAPHS = 64
NODE_DIM = 128
EDGE_DIM = 16
ACC_ROWS = 64
LANES = 16

NW = 32
N_CT = N_EDGES // 128
CT_BASE_PER_TILE = N_CT // NW
CT_EXTRA = N_CT - CT_BASE_PER_TILE * NW
CT_CHUNK = 16
E_NCHUNK = (CT_BASE_PER_TILE + 1 + CT_CHUNK - 1) // CT_CHUNK

N_PAD_BLOCKS = 80


def _sc_edge_pool(e3, a4, batch):
    mesh = plsc.VectorSubcoreMesh(core_axis_name="c", subcore_axis_name="s")
    cp = pltpu.CompilerParams()
    if "needs_layout_passes" in pltpu.CompilerParams.__dataclass_fields__:
        cp = dataclasses.replace(cp, needs_layout_passes=False)
    if "use_tc_tiling_on_sc" in pltpu.CompilerParams.__dataclass_fields__:
        cp = dataclasses.replace(cp, use_tc_tiling_on_sc=False)

    @functools.partial(
        pl.kernel,
        out_type=jax.ShapeDtypeStruct((NW, ACC_ROWS, NODE_DIM), jnp.float32),
        mesh=mesh,
        compiler_params=cp,
        scratch_types=[
            pltpu.VMEM((N_NODES,), jnp.int32),
            pltpu.VMEM((ACC_ROWS, NODE_DIM), jnp.float32),
            pltpu.VMEM((2, CT_CHUNK, 8, NODE_DIM), jnp.float32),
            pltpu.VMEM((2, CT_CHUNK, 8, NODE_DIM), jnp.float32),
            pltpu.VMEM((CT_CHUNK, NODE_DIM), jnp.int32),
            pltpu.VMEM((CT_CHUNK, NODE_DIM), jnp.int32),
            pltpu.SemaphoreType.DMA,
            pltpu.SemaphoreType.DMA,
            pltpu.SemaphoreType.DMA,
        ],
    )
    def k(e3_hbm, a4_hbm, batch_hbm, out_hbm,
          batch_v, acc, ab0, ab1, sb0, sb1,
          sem_batch, sem_e0, sem_e1):
        wid = lax.axis_index("c") * 16 + lax.axis_index("s")

        iota = lax.iota(jnp.int32, LANES)
        ones = jnp.ones((LANES,), jnp.float32)
        zeros = jnp.zeros((LANES,), jnp.float32)
        zz = jnp.zeros((LANES,), jnp.int32)
        rots = [(iota + d) & (LANES - 1) for d in range(LANES)]
        fconsts = [((r >> 3) << 14) + ((r & 7) << 7) for r in rots]

        abufs = (ab0, ab1)
        sbufs = (sb0, sb1)
        esems = (sem_e0, sem_e1)

        h_batch = pltpu.async_copy(batch_hbm, batch_v, sem_batch)

        ct_base = wid * CT_BASE_PER_TILE + jnp.minimum(wid, CT_EXTRA)
        n_ct = CT_BASE_PER_TILE + jnp.where(wid < CT_EXTRA, 1, 0)
        n_echunks = (n_ct + CT_CHUNK - 1) // CT_CHUNK

        def echunk_start_ct(c):
            return ct_base + jnp.minimum(c * CT_CHUNK, n_ct - CT_CHUNK)

        def start_echunk(c, buf):
            ct0 = echunk_start_ct(c)
            ha = pltpu.async_copy(
                a4_hbm.at[:, pl.ds(ct0, CT_CHUNK)], abufs[buf], esems[buf])
            hs = pltpu.async_copy(
                e3_hbm.at[pl.ds(ct0, CT_CHUNK), 0], sbufs[buf], esems[buf])
            return ha, hs

        he0 = start_echunk(0, 0)

        @pl.loop(0, ACC_ROWS)
        def _(r):
            for cg in range(NODE_DIM // LANES):
                acc[r, pl.ds(cg * LANES, LANES)] = zeros

        h_batch.wait()

        def e_process(c, buf):
            glo = (c * CT_CHUNK - (echunk_start_ct(c) - ct_base)) * 8

            @pl.loop(glo, CT_CHUNK * 8)
            def _(g):
                ct_l = g >> 3
                lb = (g & 7) << 4
                cbase = (ct_l << 10) + lb + iota
                s_vec = sbufs[buf][ct_l, pl.ds(lb, LANES)]
                b_vec = plsc.load_gather(batch_v, [s_vec])
                plsc.addupdate_scatter(acc, [b_vec, iota + 32], ones)
                for dd in range(0, LANES, 8):
                    vals = [
                        plsc.load_gather(
                            abufs[buf], [zz, zz, zz, fconsts[d] + cbase])
                        for d in range(dd, dd + 8)
                    ]
                    for i, d in enumerate(range(dd, dd + 8)):
                        plsc.addupdate_scatter(acc, [b_vec, rots[d]], vals[i])

        he_prev = he0
        for c in range(E_NCHUNK):
            nxt = None
            if c + 1 < E_NCHUNK:
                @pl.when(c + 1 < n_echunks)
                def _():
                    start_echunk(c + 1, (c + 1) % 2)
                nxt = (
                    pltpu.make_async_copy(
                        a4_hbm.at[:, pl.ds(0, CT_CHUNK)],
                        abufs[(c + 1) % 2], esems[(c + 1) % 2]),
                    pltpu.make_async_copy(
                        e3_hbm.at[pl.ds(0, CT_CHUNK), 0],
                        sbufs[(c + 1) % 2], esems[(c + 1) % 2]),
                )

            @pl.when(c < n_echunks)
            def _():
                he_prev[0].wait()
                he_prev[1].wait()
                e_process(c, c % 2)

            he_prev = nxt

        pltpu.sync_copy(acc, out_hbm.at[wid])

    return k(e3, a4, batch)


def _tc_node_pool(x, batch_pad):
    def body(x_ref, b_ref, xs_ref, nc_ref):
        gi = lax.broadcasted_iota(jnp.int32, (N_GRAPHS, NODE_DIM), 0)

        def step(bk, carry):
            xs, nc = carry
            bb = b_ref[bk, :].reshape(1, NODE_DIM)
            oh = (bb == gi).astype(jnp.float32)
            xblk = x_ref[pl.ds(bk * NODE_DIM, NODE_DIM), :]
            xs = xs + jnp.dot(oh, xblk, preferred_element_type=jnp.float32)
            return xs, nc + oh

        xs0 = jnp.zeros((N_GRAPHS, NODE_DIM), jnp.float32)
        xs, nc = lax.fori_loop(0, N_PAD_BLOCKS, step, (xs0, xs0))
        xs_ref[...] = xs
        nc_ref[...] = nc

    return pl.pallas_call(
        body,
        out_shape=(
            jax.ShapeDtypeStruct((N_GRAPHS, NODE_DIM), jnp.float32),
            jax.ShapeDtypeStruct((N_GRAPHS, NODE_DIM), jnp.float32),
        ),
    )(x, batch_pad)


def _tc_head(partials, xs, nc, u, W1, b1, W2, b2):
    def body(p_ref, xs_ref, nc_ref, u_ref, w1_ref, b1_ref, w2_ref, b2_ref,
             o_ref):
        p = jnp.sum(p_ref[...], axis=0)
        e_sum = p[:, 0:16]
        e_cnt = jnp.sum(p[:, 32:48], axis=1, keepdims=True)
        n_cnt = jnp.sum(nc_ref[...], axis=1, keepdims=True)
        x_mean = xs_ref[...] / jnp.maximum(n_cnt, 1.0)
        e_mean = e_sum / jnp.maximum(e_cnt, 1.0)
        uu = u_ref[...]
        w1 = w1_ref[...]
        h = (
            jnp.dot(uu, w1[0:64], preferred_element_type=jnp.float32)
            + jnp.dot(x_mean, w1[64:192], preferred_element_type=jnp.float32)
            + jnp.dot(e_mean, w1[192:208], preferred_element_type=jnp.float32)
            + b1_ref[...]
        )
        h = jnp.maximum(h, 0.0)
        o_ref[...] = (
            jnp.dot(h, w2_ref[...], preferred_element_type=jnp.float32)
            + b2_ref[...]
        )

    return pl.pallas_call(
        body,
        out_shape=jax.ShapeDtypeStruct((N_GRAPHS, W2.shape[1]), jnp.float32),
    )(partials, xs, nc, u, W1, b1.reshape(1, -1), W2, b2.reshape(1, -1))


def kernel(x, edge_index, edge_attr, u, batch, W1, b1, W2, b2):
    batch32 = batch.astype(jnp.int32)
    e3 = edge_index.astype(jnp.int32).reshape(2, N_CT, 128).transpose(1, 0, 2)
    a4 = (edge_attr.T.reshape(EDGE_DIM // 8, 8, N_CT, 128)
          .transpose(0, 2, 1, 3))
    batch_pad = jnp.pad(
        batch32, (0, N_PAD_BLOCKS * NODE_DIM - N_NODES),
        constant_values=N_GRAPHS).reshape(N_PAD_BLOCKS, NODE_DIM)
    partials = _sc_edge_pool(e3, a4, batch32)
    xs, nc = _tc_node_pool(x, batch_pad)
    return _tc_head(partials, xs, nc, u, W1, b1, W2, b2)


# speedup vs baseline: 62.0904x; 1.0049x over previous
import dataclasses
import functools

import jax
import jax.numpy as jnp
from jax import lax
from jax.experimental import pallas as pl
from jax.experimental.pallas import tpu as pltpu
from jax.experimental.pallas import tpu_sc as plsc

N_NODES = 10000
N_EDGES = 320000
N_GRAPHS = 64
NODE_DIM = 128
EDGE_DIM = 16
ACC_ROWS = 64
LANES = 16

NW = 32
N_CT = N_EDGES // 128
CT_BASE_PER_TILE = N_CT // NW
CT_EXTRA = N_CT - CT_BASE_PER_TILE * NW
CT_CHUNK = 16
E_NCHUNK = (CT_BASE_PER_TILE + 1 + CT_CHUNK - 1) // CT_CHUNK

N_PAD_BLOCKS = 80


def _sc_edge_pool(e3, a4, batch):
    mesh = plsc.VectorSubcoreMesh(core_axis_name="c", subcore_axis_name="s")
    cp = pltpu.CompilerParams()
    if "needs_layout_passes" in pltpu.CompilerParams.__dataclass_fields__:
        cp = dataclasses.replace(cp, needs_layout_passes=False)
    if "use_tc_tiling_on_sc" in pltpu.CompilerParams.__dataclass_fields__:
        cp = dataclasses.replace(cp, use_tc_tiling_on_sc=False)

    @functools.partial(
        pl.kernel,
        out_type=jax.ShapeDtypeStruct((NW, ACC_ROWS, NODE_DIM), jnp.float32),
        mesh=mesh,
        compiler_params=cp,
        scratch_types=[
            pltpu.VMEM((N_NODES,), jnp.int32),
            pltpu.VMEM((ACC_ROWS, NODE_DIM), jnp.float32),
            pltpu.VMEM((2, CT_CHUNK, 8, NODE_DIM), jnp.float32),
            pltpu.VMEM((2, CT_CHUNK, 8, NODE_DIM), jnp.float32),
            pltpu.VMEM((CT_CHUNK, NODE_DIM), jnp.int32),
            pltpu.VMEM((CT_CHUNK, NODE_DIM), jnp.int32),
            pltpu.SemaphoreType.DMA,
            pltpu.SemaphoreType.DMA,
            pltpu.SemaphoreType.DMA,
        ],
    )
    def k(e3_hbm, a4_hbm, batch_hbm, out_hbm,
          batch_v, acc, ab0, ab1, sb0, sb1,
          sem_batch, sem_e0, sem_e1):
        wid = lax.axis_index("c") * 16 + lax.axis_index("s")

        iota = lax.iota(jnp.int32, LANES)
        ones = jnp.ones((LANES,), jnp.float32)
        zeros = jnp.zeros((LANES,), jnp.float32)
        zz = jnp.zeros((LANES,), jnp.int32)
        rots = [(iota + d) & (LANES - 1) for d in range(LANES)]
        fconsts = [((r >> 3) << 14) + ((r & 7) << 7) for r in rots]

        abufs = (ab0, ab1)
        sbufs = (sb0, sb1)
        esems = (sem_e0, sem_e1)

        h_batch = pltpu.async_copy(batch_hbm, batch_v, sem_batch)

        ct_base = wid * CT_BASE_PER_TILE + jnp.minimum(wid, CT_EXTRA)
        n_ct = CT_BASE_PER_TILE + jnp.where(wid < CT_EXTRA, 1, 0)
        n_echunks = (n_ct + CT_CHUNK - 1) // CT_CHUNK

        def echunk_start_ct(c):
            return ct_base + jnp.minimum(c * CT_CHUNK, n_ct - CT_CHUNK)

        def start_echunk(c, buf):
            ct0 = echunk_start_ct(c)
            ha = pltpu.async_copy(
                a4_hbm.at[:, pl.ds(ct0, CT_CHUNK)], abufs[buf], esems[buf])
            hs = pltpu.async_copy(
                e3_hbm.at[pl.ds(ct0, CT_CHUNK), 0], sbufs[buf], esems[buf])
            return ha, hs

        he0 = start_echunk(0, 0)

        @pl.loop(0, ACC_ROWS)
        def _(r):
            for cg in range(NODE_DIM // LANES):
                acc[r, pl.ds(cg * LANES, LANES)] = zeros

        h_batch.wait()

        def e_process(c, buf):
            glo = (c * CT_CHUNK - (echunk_start_ct(c) - ct_base)) * 8

            @pl.loop(glo, CT_CHUNK * 8)
            def _(g):
                ct_l = g >> 3
                lb = (g & 7) << 4
                cbase = (ct_l << 10) + lb + iota
                s_vec = sbufs[buf][ct_l, pl.ds(lb, LANES)]
                b_vec = plsc.load_gather(batch_v, [s_vec])
                plsc.addupdate_scatter(acc, [b_vec, iota + 32], ones)
                for dd in range(0, LANES, 8):
                    vals = [
                        plsc.load_gather(
                            abufs[buf], [zz, zz, zz, fconsts[d] + cbase])
                        for d in range(dd, dd + 8)
                    ]
                    for i, d in enumerate(range(dd, dd + 8)):
                        plsc.addupdate_scatter(acc, [b_vec, rots[d]], vals[i])

        he_prev = he0
        for c in range(E_NCHUNK):
            nxt = None
            if c + 1 < E_NCHUNK:
                @pl.when(c + 1 < n_echunks)
                def _():
                    start_echunk(c + 1, (c + 1) % 2)
                nxt = (
                    pltpu.make_async_copy(
                        a4_hbm.at[:, pl.ds(0, CT_CHUNK)],
                        abufs[(c + 1) % 2], esems[(c + 1) % 2]),
                    pltpu.make_async_copy(
                        e3_hbm.at[pl.ds(0, CT_CHUNK), 0],
                        sbufs[(c + 1) % 2], esems[(c + 1) % 2]),
                )

            @pl.when(c < n_echunks)
            def _():
                he_prev[0].wait()
                he_prev[1].wait()
                e_process(c, c % 2)

            he_prev = nxt

        pltpu.sync_copy(acc, out_hbm.at[wid])

    return k(e3, a4, batch)


def _tc_node_pool(x, batch_pad):
    def body(x_ref, b_ref, xs_ref, nc_ref):
        gi = lax.broadcasted_iota(jnp.int32, (N_GRAPHS, NODE_DIM), 0)

        def step(bk, carry):
            xs, nc = carry
            bb = b_ref[bk, :].reshape(1, NODE_DIM)
            oh = (bb == gi).astype(jnp.float32)
            xblk = x_ref[pl.ds(bk * NODE_DIM, NODE_DIM), :]
            xs = xs + jnp.dot(oh, xblk, preferred_element_type=jnp.float32,
                              precision=lax.Precision.HIGHEST)
            return xs, nc + oh

        xs0 = jnp.zeros((N_GRAPHS, NODE_DIM), jnp.float32)
        xs, nc = lax.fori_loop(0, N_PAD_BLOCKS, step, (xs0, xs0))
        xs_ref[...] = xs
        nc_ref[...] = nc

    return pl.pallas_call(
        body,
        out_shape=(
            jax.ShapeDtypeStruct((N_GRAPHS, NODE_DIM), jnp.float32),
            jax.ShapeDtypeStruct((N_GRAPHS, NODE_DIM), jnp.float32),
        ),
    )(x, batch_pad)


def _tc_head(partials, xs, nc, u, W1, b1, W2, b2):
    def body(p_ref, xs_ref, nc_ref, u_ref, w1_ref, b1_ref, w2_ref, b2_ref,
             o_ref):
        p = jnp.sum(p_ref[...], axis=0)
        e_sum = p[:, 0:16]
        e_cnt = jnp.sum(p[:, 32:48], axis=1, keepdims=True)
        n_cnt = jnp.sum(nc_ref[...], axis=1, keepdims=True)
        x_mean = xs_ref[...] / jnp.maximum(n_cnt, 1.0)
        e_mean = e_sum / jnp.maximum(e_cnt, 1.0)
        uu = u_ref[...]
        w1 = w1_ref[...]
        h = (
            jnp.dot(uu, w1[0:64], preferred_element_type=jnp.float32)
            + jnp.dot(x_mean, w1[64:192], preferred_element_type=jnp.float32)
            + jnp.dot(e_mean, w1[192:208], preferred_element_type=jnp.float32)
            + b1_ref[...]
        )
        h = jnp.maximum(h, 0.0)
        o_ref[...] = (
            jnp.dot(h, w2_ref[...], preferred_element_type=jnp.float32)
            + b2_ref[...]
        )

    return pl.pallas_call(
        body,
        out_shape=jax.ShapeDtypeStruct((N_GRAPHS, W2.shape[1]), jnp.float32),
    )(partials, xs, nc, u, W1, b1.reshape(1, -1), W2, b2.reshape(1, -1))


def kernel(x, edge_index, edge_attr, u, batch, W1, b1, W2, b2):
    batch32 = batch.astype(jnp.int32)
    e3 = edge_index.astype(jnp.int32).reshape(2, N_CT, 128).transpose(1, 0, 2)
    a4 = (edge_attr.T.reshape(EDGE_DIM // 8, 8, N_CT, 128)
          .transpose(0, 2, 1, 3))
    batch_pad = jnp.pad(
        batch32, (0, N_PAD_BLOCKS * NODE_DIM - N_NODES),
        constant_values=N_GRAPHS).reshape(N_PAD_BLOCKS, NODE_DIM)
    partials = _sc_edge_pool(e3, a4, batch32)
    xs, nc = _tc_node_pool(x, batch_pad)
    return _tc_head(partials, xs, nc, u, W1, b1, W2, b2)
